# baseline (speedup 1.0000x reference)
import functools

import jax
import jax.numpy as jnp
from jax import lax
from jax.experimental import pallas as pl
from jax.experimental.pallas import tpu as pltpu
from jax.experimental.pallas import tpu_sc as plsc

LANES = 16
NCORES = 2
NSUB = 16
B = 128
SUP = 8
NBUF = 8
KG = 2
EBLK = 8


def _matmul_body(n, n_pad, dh, x_ref, w_ref, o_ref):
    s = jnp.dot(x_ref[...], w_ref[...], preferred_element_type=jnp.float32)
    zeros = jnp.zeros((n_pad - n, dh), jnp.float32)
    for c in range(NCORES):
        o_ref[c, pl.ds(0, n)] = s[:, c * dh:(c + 1) * dh]
        o_ref[c, pl.ds(n, n_pad - n)] = zeros


def _combine_body(n, p_ref, b_ref, o_ref):
    o_ref[...] = jnp.concatenate([p_ref[0, :n], p_ref[1, :n]], axis=1) \
        + b_ref[...]


def _make_spmm(n_pad, dh, g):
    rows_per_tile = n_pad // NSUB
    nchunks = rows_per_tile // B
    vecs = dh // LANES
    supers = g // SUP
    assert g % (2 * SUP) == 0 and supers >= 2

    mesh = plsc.VectorSubcoreMesh(core_axis_name="c", subcore_axis_name="s")

    @functools.partial(
        pl.kernel,
        out_type=jax.ShapeDtypeStruct((NCORES, n_pad, dh), jnp.float32),
        mesh=mesh,
        scratch_types=[
            pltpu.VMEM((2, SUP, B), jnp.int32),
            pltpu.VMEM((2, SUP, B), jnp.int32),
            pltpu.VMEM((2, SUP, B), jnp.float32),
            pltpu.VMEM((NBUF, B, dh), jnp.float32),
            pltpu.VMEM_SHARED((n_pad, dh), jnp.float32),
            pltpu.SemaphoreType.DMA((2,)),
            pltpu.SemaphoreType.DMA((2,)),
            pltpu.SemaphoreType.DMA((2,)),
            pltpu.SemaphoreType.DMA((NBUF,)),
            pltpu.SemaphoreType.DMA((NBUF,)),
        ],
        compiler_params=pltpu.CompilerParams(use_tc_tiling_on_sc=False),
    )
    def spmm(support_hbm, src_hbm, dst_hbm, w_hbm, out_hbm,
             sring, dring, wring, rows, acc, stss, stds, stws, gsem, ssem):
        cid = lax.axis_index("c")
        sid = lax.axis_index("s")

        zero = jnp.zeros((LANES,), jnp.float32)

        def zero_row(i, _):
            for j in range(vecs):
                rows[0, i, pl.ds(LANES * j, LANES)] = zero
            return 0

        lax.fori_loop(0, B, zero_row, 0)
        rbase = sid * rows_per_tile
        for k in range(nchunks):
            pltpu.sync_copy(rows.at[0], acc.at[pl.ds(rbase + B * k, B)])
        plsc.subcore_barrier()

        def stage_start(si, sl):
            bs = pl.ds(si * SUP, SUP)
            pltpu.async_copy(src_hbm.at[cid, sid, bs], sring.at[sl],
                             stss.at[sl])
            pltpu.async_copy(dst_hbm.at[sid, bs], dring.at[sl], stds.at[sl])
            pltpu.async_copy(w_hbm.at[sid, bs], wring.at[sl], stws.at[sl])

        def stage_wait(si, sl):
            bs = pl.ds(si * SUP, SUP)
            pltpu.make_async_copy(src_hbm.at[cid, sid, bs], sring.at[sl],
                                  stss.at[sl]).wait()
            pltpu.make_async_copy(dst_hbm.at[sid, bs], dring.at[sl],
                                  stds.at[sl]).wait()
            pltpu.make_async_copy(w_hbm.at[sid, bs], wring.at[sl],
                                  stws.at[sl]).wait()

        def gather_start(stsl, b, k):
            pltpu.async_copy(support_hbm.at[sring.at[stsl, b]], rows.at[k],
                             gsem.at[k])

        def gather_wait(stsl, b, k):
            pltpu.make_async_copy(support_hbm.at[sring.at[stsl, b]],
                                  rows.at[k], gsem.at[k]).wait()

        def scatter_start(stsl, b, k):
            pltpu.async_copy(rows.at[k], acc.at[dring.at[stsl, b]],
                             ssem.at[k], add=True)

        def scatter_wait(stsl, b, k):
            pltpu.make_async_copy(rows.at[k], acc.at[dring.at[stsl, b]],
                                  ssem.at[k]).wait()

        def scale(stsl, b, k):
            def scale_group(q, _):
                base = LANES * q
                wvec = wring[stsl, b, pl.ds(base, LANES)]
                for el0 in range(0, LANES, EBLK):
                    ws = [wvec[el0 + i] for i in range(EBLK)]
                    vals = [rows[k, base + el0 + i, pl.ds(LANES * j, LANES)]
                            for i in range(EBLK) for j in range(vecs)]
                    for i in range(EBLK):
                        for j in range(vecs):
                            c = pl.ds(LANES * j, LANES)
                            rows[k, base + el0 + i, c] = \
                                vals[i * vecs + j] * ws[i]
                return 0

            lax.fori_loop(0, B // LANES, scale_group, 0)

        def super_steps(si, par):
            nxt = 1 - par
            for k in range(SUP):
                gi = si * SUP + k
                kk = (k + KG) % NBUF
                if k + KG < SUP:
                    @pl.when(gi + KG < g)
                    def _(k=k, kk=kk, par=par):
                        gather_start(par, k + KG, kk)
                else:
                    @pl.when(gi + KG < g)
                    def _(k=k, kk=kk, nxt=nxt, si=si):
                        if k + KG == SUP:
                            stage_wait(si + 1, nxt)
                        gather_start(nxt, k + KG - SUP, kk)

                gather_wait(par, k, k)
                scale(par, k, k)
                s_prev = (k + NBUF - 1) % NBUF

                @pl.when(gi >= 1)
                def _(k=k, s_prev=s_prev, par=par, nxt=nxt):
                    if k == 0:
                        scatter_wait(nxt, SUP - 1, s_prev)
                    else:
                        scatter_wait(par, k - 1, s_prev)

                if k == 1:
                    @pl.when(si + 1 < supers)
                    def _(si=si, nxt=nxt):
                        stage_start(si + 1, nxt)

                scatter_start(par, k, k)

        stage_start(jnp.int32(0), 0)
        stage_wait(jnp.int32(0), 0)
        for b0 in range(min(KG, SUP)):
            gather_start(0, b0, b0)

        def outer(t, _):
            super_steps(t * 2, 0)
            super_steps(t * 2 + 1, 1)
            return 0

        lax.fori_loop(0, supers // 2, outer, 0)
        scatter_wait((supers - 1) % 2, SUP - 1, NBUF - 1)

        plsc.subcore_barrier()
        for k in range(nchunks):
            sl = pl.ds(rbase + B * k, B)
            pltpu.sync_copy(acc.at[sl], out_hbm.at[cid, sl])

    return spmm


def kernel(x, edge_index, edge_weight, W, b):
    n, d_in = x.shape
    d = W.shape[1]
    dh = d // NCORES
    e = edge_weight.shape[0]
    n_pad = -(-n // (NSUB * B)) * (NSUB * B)

    support = pl.pallas_call(
        functools.partial(_matmul_body, n, n_pad, dh),
        out_shape=jax.ShapeDtypeStruct((NCORES, n_pad, dh), jnp.float32),
    )(x, W)

    per_t = -(-e // NSUB)
    g = -(-per_t // (B * 2 * SUP)) * (2 * SUP)
    pad = NSUB * g * B - e
    src = jnp.pad(edge_index[0], (0, pad)).reshape(NSUB, g, B)
    dst = jnp.pad(edge_index[1], (0, pad)).reshape(NSUB, g, B)
    wdat = jnp.pad(edge_weight, (0, pad)).reshape(NSUB, g, B)
    coff = (jnp.arange(NCORES, dtype=jnp.int32) * n_pad)[:, None, None, None]
    src2 = src[None] + coff

    partials = _make_spmm(n_pad, dh, g)(
        support.reshape(NCORES * n_pad, dh), src2, dst, wdat)

    out = pl.pallas_call(
        functools.partial(_combine_body, n),
        out_shape=jax.ShapeDtypeStruct((n, d), jnp.float32),
    )(partials, b.reshape(1, d))
    return out


# speedup vs baseline: 1.7758x; 1.7758x over previous
import functools

import jax
import jax.numpy as jnp
from jax import lax
from jax.experimental import pallas as pl
from jax.experimental.pallas import tpu as pltpu
from jax.experimental.pallas import tpu_sc as plsc

LANES = 16
NCORES = 2
NSUB = 16
B = 128
NBUF = 6
KG = 2


def _matmul_body(n, n_pad, dh, x_ref, w_ref, o_ref):
    s = jnp.dot(x_ref[...], w_ref[...], preferred_element_type=jnp.float32)
    zeros = jnp.zeros((n_pad - n, dh), jnp.float32)
    for c in range(NCORES):
        o_ref[c, pl.ds(0, n)] = s[:, c * dh:(c + 1) * dh]
        o_ref[c, pl.ds(n, n_pad - n)] = zeros


def _combine_body(n, p_ref, b_ref, o_ref):
    o_ref[...] = jnp.concatenate([p_ref[0, :n], p_ref[1, :n]], axis=1) \
        + b_ref[...]


def _make_spmm(n_pad, dh, g):
    rows_per_tile = n_pad // NSUB
    nchunks = rows_per_tile // B
    vecs = dh // LANES

    mesh = plsc.VectorSubcoreMesh(core_axis_name="c", subcore_axis_name="s")

    @functools.partial(
        pl.kernel,
        out_type=jax.ShapeDtypeStruct((NCORES, n_pad, dh), jnp.float32),
        mesh=mesh,
        scratch_types=[
            pltpu.VMEM((NBUF, B), jnp.int32),
            pltpu.VMEM((NBUF, B), jnp.int32),
            pltpu.VMEM((NBUF, B), jnp.float32),
            pltpu.VMEM((NBUF, B, dh), jnp.float32),
            pltpu.VMEM_SHARED((n_pad, dh), jnp.float32),
            pltpu.SemaphoreType.DMA((NBUF,)),
            pltpu.SemaphoreType.DMA((NBUF,)),
            pltpu.SemaphoreType.DMA((NBUF,)),
            pltpu.SemaphoreType.DMA((NBUF,)),
            pltpu.SemaphoreType.DMA((NBUF,)),
        ],
        compiler_params=pltpu.CompilerParams(use_tc_tiling_on_sc=False),
    )
    def spmm(support_hbm, src_hbm, dst_hbm, w_hbm, out_hbm,
             sring, dring, wring, rows, acc, isem, dsem, wsem, gsem, ssem):
        cid = lax.axis_index("c")
        sid = lax.axis_index("s")
        zero = jnp.zeros((LANES,), jnp.float32)

        def zero_row(i, _):
            for j in range(vecs):
                rows[0, i, pl.ds(LANES * j, LANES)] = zero
            return 0

        lax.fori_loop(0, B, zero_row, 0)
        rbase = sid * rows_per_tile
        for k in range(nchunks):
            pltpu.sync_copy(rows.at[0], acc.at[pl.ds(rbase + B * k, B)])
        plsc.subcore_barrier()

        def idx_start(gi, sl):
            pltpu.async_copy(src_hbm.at[cid, sid, gi], sring.at[sl],
                             isem.at[sl])
            pltpu.async_copy(dst_hbm.at[sid, gi], dring.at[sl], dsem.at[sl])
            pltpu.async_copy(w_hbm.at[sid, gi], wring.at[sl], wsem.at[sl])

        def idx_wait(gi, sl):
            pltpu.make_async_copy(src_hbm.at[cid, sid, gi], sring.at[sl],
                                  isem.at[sl]).wait()
            pltpu.make_async_copy(dst_hbm.at[sid, gi], dring.at[sl],
                                  dsem.at[sl]).wait()
            pltpu.make_async_copy(w_hbm.at[sid, gi], wring.at[sl],
                                  wsem.at[sl]).wait()

        def gather_start(sl):
            pltpu.async_copy(support_hbm.at[sring.at[sl]], rows.at[sl],
                             gsem.at[sl])

        def gather_wait(sl):
            pltpu.make_async_copy(support_hbm.at[sring.at[sl]],
                                  rows.at[sl], gsem.at[sl]).wait()

        def scatter_start(sl):
            pltpu.async_copy(rows.at[sl], acc.at[dring.at[sl]],
                             ssem.at[sl], add=True)

        def scatter_wait(sl):
            pltpu.make_async_copy(rows.at[sl], acc.at[dring.at[sl]],
                                  ssem.at[sl]).wait()

        EBLK = 8

        def scale(sl):
            def scale_group(q, _):
                base = LANES * q
                wvec = wring[sl, pl.ds(base, LANES)]
                for el0 in range(0, LANES, EBLK):
                    ws = [wvec[el0 + i] for i in range(EBLK)]
                    vals = [rows[sl, base + el0 + i, pl.ds(LANES * j, LANES)]
                            for i in range(EBLK) for j in range(vecs)]
                    for i in range(EBLK):
                        for j in range(vecs):
                            c = pl.ds(LANES * j, LANES)
                            rows[sl, base + el0 + i, c] = \
                                vals[i * vecs + j] * ws[i]
                return 0

            lax.fori_loop(0, B // LANES, scale_group, 0)

        def step(gi, k):
            @pl.when(gi + KG < g)
            def _():
                idx_wait(gi + KG, (k + KG) % NBUF)
                gather_start((k + KG) % NBUF)

            gather_wait(k)
            scale(k)
            s_prev = (k + NBUF - 1) % NBUF

            @pl.when(gi >= 1)
            def _():
                scatter_wait(s_prev)

            @pl.when(gi + NBUF - 1 < g)
            def _():
                idx_start(gi + NBUF - 1, s_prev)

            scatter_start(k)

        for b0 in range(min(NBUF - 1, g)):
            idx_start(b0, b0)
        for b0 in range(min(KG, g)):
            idx_wait(b0, b0)
            gather_start(b0)

        g_main = g // NBUF * NBUF

        def outer(t, _):
            for k in range(NBUF):
                step(t * NBUF + k, k)
            return 0

        lax.fori_loop(0, g_main // NBUF, outer, 0)
        for gi in range(g_main, g):
            step(jnp.int32(gi), gi % NBUF)
        scatter_wait((g - 1) % NBUF)

        plsc.subcore_barrier()
        for k in range(nchunks):
            sl = pl.ds(rbase + B * k, B)
            pltpu.sync_copy(acc.at[sl], out_hbm.at[cid, sl])

    return spmm


def kernel(x, edge_index, edge_weight, W, b):
    n, d_in = x.shape
    d = W.shape[1]
    dh = d // NCORES
    e = edge_weight.shape[0]
    n_pad = -(-n // (NSUB * B)) * (NSUB * B)

    support = pl.pallas_call(
        functools.partial(_matmul_body, n, n_pad, dh),
        out_shape=jax.ShapeDtypeStruct((NCORES, n_pad, dh), jnp.float32),
    )(x, W)

    per_t = -(-e // NSUB)
    g = -(-per_t // B)
    pad = NSUB * g * B - e
    src = jnp.pad(edge_index[0], (0, pad)).reshape(NSUB, g, B)
    dst = jnp.pad(edge_index[1], (0, pad)).reshape(NSUB, g, B)
    wdat = jnp.pad(edge_weight, (0, pad)).reshape(NSUB, g, B)
    coff = (jnp.arange(NCORES, dtype=jnp.int32) * n_pad)[:, None, None, None]
    src2 = src[None] + coff

    partials = _make_spmm(n_pad, dh, g)(
        support.reshape(NCORES * n_pad, dh), src2, dst, wdat)

    out = pl.pallas_call(
        functools.partial(_combine_body, n),
        out_shape=jax.ShapeDtypeStruct((n, d), jnp.float32),
    )(partials, b.reshape(1, d))
    return out
